# baseline (speedup 1.0000x reference)
import functools

import jax
import jax.numpy as jnp
from jax.experimental import pallas as pl

NUM_HEADS = 16
HEAD_DIM = 128


def _matmul_kernel(x_ref, w_ref, o_ref):
    o_ref[...] = jax.lax.dot_general(
        x_ref[...], w_ref[...],
        dimension_numbers=(((1,), (1,)), ((), ())),
        preferred_element_type=jnp.float32)


def _matmul_bias_kernel(x_ref, w_ref, b_ref, o_ref):
    o_ref[...] = jax.lax.dot_general(
        x_ref[...], w_ref[...],
        dimension_numbers=(((1,), (1,)), ((), ())),
        preferred_element_type=jnp.float32) + b_ref[...]


def _matmul_t(x, w, bm, bn, bias=None, interpret=False):
    m, k = x.shape
    n, k2 = w.shape
    assert k == k2 and m % bm == 0 and n % bn == 0
    grid = (m // bm, n // bn)
    x_spec = pl.BlockSpec((bm, k), lambda i, j: (i, 0))
    w_spec = pl.BlockSpec((bn, k), lambda i, j: (j, 0))
    o_spec = pl.BlockSpec((bm, bn), lambda i, j: (i, j))
    out_type = jax.ShapeDtypeStruct((m, n), jnp.float32)
    if bias is None:
        return pl.pallas_call(
            _matmul_kernel, grid=grid,
            in_specs=[x_spec, w_spec], out_specs=o_spec,
            out_shape=out_type, interpret=interpret)(x, w)
    b_spec = pl.BlockSpec((1, bn), lambda i, j: (0, j))
    return pl.pallas_call(
        _matmul_bias_kernel, grid=grid,
        in_specs=[x_spec, w_spec, b_spec], out_specs=o_spec,
        out_shape=out_type, interpret=interpret)(x, w, bias.reshape(1, n))


def _flash_kernel(q_ref, k_ref, v_ref, o_ref, *, bq, bk, scale):
    qi = pl.program_id(1)
    q = q_ref[0] * scale

    def attend(carry, j, masked):
        m, l, acc = carry
        kb = k_ref[0, pl.ds(j * bk, bk), :]
        s = jax.lax.dot_general(
            q, kb, dimension_numbers=(((1,), (1,)), ((), ())),
            preferred_element_type=jnp.float32)
        if masked:
            rows = jax.lax.broadcasted_iota(jnp.int32, (bq, bk), 0)
            cols = jax.lax.broadcasted_iota(jnp.int32, (bq, bk), 1)
            s = jnp.where(rows >= cols, s, -jnp.inf)
        m_new = jnp.maximum(m, jnp.max(s, axis=1, keepdims=True))
        p = jnp.exp(s - m_new)
        alpha = jnp.exp(m - m_new)
        l = l * alpha + jnp.sum(p, axis=1, keepdims=True)
        vb = v_ref[0, pl.ds(j * bk, bk), :]
        acc = acc * alpha + jnp.dot(p, vb, preferred_element_type=jnp.float32)
        return m_new, l, acc

    carry = (jnp.full((bq, 1), -jnp.inf, jnp.float32),
             jnp.zeros((bq, 1), jnp.float32),
             jnp.zeros((bq, HEAD_DIM), jnp.float32))
    carry = jax.lax.fori_loop(
        0, qi, lambda j, c: attend(c, j, masked=False), carry)
    _, l, acc = attend(carry, qi, masked=True)
    o_ref[0] = acc / l


def _flash_attention(qkv, bq, bk, interpret=False):
    b, s, three_hidden = qkv.shape
    hidden = three_hidden // 3
    scale = 1.0 / (HEAD_DIM ** 0.5)
    grid = (b * NUM_HEADS, s // bq)
    q_spec = pl.BlockSpec(
        (1, bq, HEAD_DIM),
        lambda bh, qi: (bh // NUM_HEADS, qi, bh % NUM_HEADS))
    k_spec = pl.BlockSpec(
        (1, s, HEAD_DIM),
        lambda bh, qi: (bh // NUM_HEADS, 0, NUM_HEADS + bh % NUM_HEADS))
    v_spec = pl.BlockSpec(
        (1, s, HEAD_DIM),
        lambda bh, qi: (bh // NUM_HEADS, 0, 2 * NUM_HEADS + bh % NUM_HEADS))
    o_spec = pl.BlockSpec(
        (1, bq, HEAD_DIM),
        lambda bh, qi: (bh // NUM_HEADS, qi, bh % NUM_HEADS))
    return pl.pallas_call(
        functools.partial(_flash_kernel, bq=bq, bk=bk, scale=scale),
        grid=grid,
        in_specs=[q_spec, k_spec, v_spec],
        out_specs=o_spec,
        out_shape=jax.ShapeDtypeStruct((b, s, hidden), jnp.float32),
        interpret=interpret)(qkv, qkv, qkv)


def kernel(x, Wq, Wk, Wv, Wo, bo, interpret=False):
    b, s, hidden = x.shape
    wc = jnp.concatenate([Wq, Wk, Wv], axis=0)
    x2 = x.reshape(b * s, hidden)
    qkv = _matmul_t(x2, wc, bm=1024, bn=512, interpret=interpret)
    qkv = qkv.reshape(b, s, 3 * hidden)
    attn = _flash_attention(qkv, bq=256, bk=256, interpret=interpret)
    out = _matmul_t(attn.reshape(b * s, hidden), Wo, bm=1024, bn=512,
                    bias=bo, interpret=interpret)
    return out.reshape(b, s, hidden)


# speedup vs baseline: 1.4601x; 1.4601x over previous
import functools

import jax
import jax.numpy as jnp
from jax.experimental import pallas as pl
from jax.experimental.pallas import tpu as pltpu

NUM_HEADS = 16
HEAD_DIM = 128


def _matmul_kernel(x_ref, w_ref, o_ref):
    o_ref[...] = jax.lax.dot_general(
        x_ref[...], w_ref[...],
        dimension_numbers=(((1,), (1,)), ((), ())),
        preferred_element_type=jnp.float32)


def _matmul_bias_kernel(x_ref, w_ref, b_ref, o_ref):
    o_ref[...] = jax.lax.dot_general(
        x_ref[...], w_ref[...],
        dimension_numbers=(((1,), (1,)), ((), ())),
        preferred_element_type=jnp.float32) + b_ref[...]


def _matmul_t(x, w, bm, bn, bias=None, interpret=False):
    m, k = x.shape
    n, k2 = w.shape
    assert k == k2 and m % bm == 0 and n % bn == 0
    grid = (m // bm, n // bn)
    x_spec = pl.BlockSpec((bm, k), lambda i, j: (i, 0))
    w_spec = pl.BlockSpec((bn, k), lambda i, j: (j, 0))
    o_spec = pl.BlockSpec((bm, bn), lambda i, j: (i, j))
    out_type = jax.ShapeDtypeStruct((m, n), jnp.float32)
    if bias is None:
        return pl.pallas_call(
            _matmul_kernel, grid=grid,
            in_specs=[x_spec, w_spec], out_specs=o_spec,
            out_shape=out_type, interpret=interpret)(x, w)
    b_spec = pl.BlockSpec((1, bn), lambda i, j: (0, j))
    return pl.pallas_call(
        _matmul_bias_kernel, grid=grid,
        in_specs=[x_spec, w_spec, b_spec], out_specs=o_spec,
        out_shape=out_type, interpret=interpret)(x, w, bias.reshape(1, n))


def _flash_kernel(q_ref, k_ref, v_ref, o_ref, s_scr, *, bq, bk, scale):
    qi = pl.program_id(1)
    nb = qi + 1
    q = q_ref[0] * scale

    rows = jax.lax.broadcasted_iota(jnp.int32, (bq, bk), 0)
    cols = jax.lax.broadcasted_iota(jnp.int32, (bq, bk), 1)

    def pass_a(j, m):
        kb = k_ref[0, pl.ds(j * bk, bk), :]
        s = jax.lax.dot_general(
            q, kb, dimension_numbers=(((1,), (1,)), ((), ())),
            preferred_element_type=jnp.float32)
        s = jnp.where((j * bk + cols) <= (qi * bq + rows), s, -jnp.inf)
        s_scr[:, pl.ds(j * bk, bk)] = s
        return jnp.maximum(m, jnp.max(s, axis=1, keepdims=True))

    m = jax.lax.fori_loop(
        0, nb, pass_a, jnp.full((bq, 1), -jnp.inf, jnp.float32))

    def pass_b(j, carry):
        l, acc = carry
        p = jnp.exp(s_scr[:, pl.ds(j * bk, bk)] - m)
        l = l + jnp.sum(p, axis=1, keepdims=True)
        vb = v_ref[0, pl.ds(j * bk, bk), :]
        acc = acc + jnp.dot(p, vb, preferred_element_type=jnp.float32)
        return l, acc

    l, acc = jax.lax.fori_loop(
        0, nb, pass_b, (jnp.zeros((bq, 1), jnp.float32),
                        jnp.zeros((bq, HEAD_DIM), jnp.float32)))
    o_ref[0] = acc / l


def _flash_attention(qkv, bq, bk, interpret=False):
    b, s, three_hidden = qkv.shape
    hidden = three_hidden // 3
    scale = 1.0 / (HEAD_DIM ** 0.5)
    grid = (b * NUM_HEADS, s // bq)
    q_spec = pl.BlockSpec(
        (1, bq, HEAD_DIM),
        lambda bh, qi: (bh // NUM_HEADS, qi, bh % NUM_HEADS))
    k_spec = pl.BlockSpec(
        (1, s, HEAD_DIM),
        lambda bh, qi: (bh // NUM_HEADS, 0, NUM_HEADS + bh % NUM_HEADS))
    v_spec = pl.BlockSpec(
        (1, s, HEAD_DIM),
        lambda bh, qi: (bh // NUM_HEADS, 0, 2 * NUM_HEADS + bh % NUM_HEADS))
    o_spec = pl.BlockSpec(
        (1, bq, HEAD_DIM),
        lambda bh, qi: (bh // NUM_HEADS, qi, bh % NUM_HEADS))
    return pl.pallas_call(
        functools.partial(_flash_kernel, bq=bq, bk=bk, scale=scale),
        grid=grid,
        in_specs=[q_spec, k_spec, v_spec],
        out_specs=o_spec,
        out_shape=jax.ShapeDtypeStruct((b, s, hidden), jnp.float32),
        scratch_shapes=[pltpu.VMEM((bq, s), jnp.float32)],
        interpret=interpret)(qkv, qkv, qkv)


def kernel(x, Wq, Wk, Wv, Wo, bo, interpret=False):
    b, s, hidden = x.shape
    wc = jnp.concatenate([Wq, Wk, Wv], axis=0)
    x2 = x.reshape(b * s, hidden)
    qkv = _matmul_t(x2, wc, bm=1024, bn=512, interpret=interpret)
    qkv = qkv.reshape(b, s, 3 * hidden)
    attn = _flash_attention(qkv, bq=512, bk=512, interpret=interpret)
    out = _matmul_t(attn.reshape(b * s, hidden), Wo, bm=1024, bn=512,
                    bias=bo, interpret=interpret)
    return out.reshape(b, s, hidden)
